# Initial kernel scaffold; baseline (speedup 1.0000x reference)
import functools

import jax
import jax.numpy as jnp
from jax import lax
from jax.experimental import pallas as pl
from jax.experimental.pallas import tpu as pltpu
from jax.experimental.pallas import tpu_sc as plsc


def _sc_gather(table, idx):
    V, D = table.shape
    N = idx.shape[0]
    info = plsc.get_sparse_core_info()
    NC, NS = info.num_cores, info.num_subcores
    NW = NC * NS
    assert N % NW == 0
    per_w = N // NW
    K = 128
    assert per_w % K == 0
    n_chunks = per_w // K
    NBUF = 5 if n_chunks % 5 == 0 else (4 if n_chunks % 4 == 0 else 2)
    assert n_chunks % NBUF == 0
    n_groups = n_chunks // NBUF

    mesh = plsc.VectorSubcoreMesh(core_axis_name="c", subcore_axis_name="s")

    @functools.partial(
        pl.kernel,
        mesh=mesh,
        out_type=jax.ShapeDtypeStruct((N, D), jnp.float32),
        scratch_types=(
            [pltpu.VMEM((per_w,), jnp.int32)]
            + [pltpu.VMEM((K, D), jnp.float32) for _ in range(NBUF)]
            + [pltpu.SemaphoreType.DMA for _ in range(2 * NBUF)]
        ),
    )
    def k(table_hbm, idx_hbm, out_hbm, *scr):
        idx_v = scr[0]
        bufs = scr[1:1 + NBUF]
        gsems = scr[1 + NBUF:1 + 2 * NBUF]
        wsems = scr[1 + 2 * NBUF:1 + 3 * NBUF]
        wid = lax.axis_index("s") * NC + lax.axis_index("c")
        base = wid * per_w
        pltpu.sync_copy(idx_hbm.at[pl.ds(base, per_w)], idx_v)

        def group(g, carry):
            off = g * (NBUF * K)
            gathers = []
            for b in range(NBUF):
                cp = pltpu.make_async_copy(
                    table_hbm.at[idx_v.at[pl.ds(off + b * K, K)]],
                    bufs[b], gsems[b])
                cp.start()
                gathers.append(cp)
            writes = []
            for b in range(NBUF):
                gathers[b].wait()
                wp = pltpu.make_async_copy(
                    bufs[b], out_hbm.at[pl.ds(base + off + b * K, K)],
                    wsems[b])
                wp.start()
                writes.append(wp)
            for b in range(NBUF):
                writes[b].wait()
            return carry

        lax.fori_loop(0, n_groups, group, 0, unroll=False)

    return k(table, idx)


def _crf_body(L, B, C,
              embed_ref, lab_ref, W_ref, WT_ref, transT_ref,
              start_ref, end_ref, brow_ref, bcol_ref,
              logits_ref, loss_ref,
              alpha, gold, ohprev):
    t = pl.program_id(0)
    E = embed_ref[0]
    HI = jax.lax.Precision.HIGHEST
    P = lax.dot_general(E, W_ref[...], (((1,), (0,)), ((), ())),
                        precision=HI) + brow_ref[...]
    logits_ref[0] = P
    PT = lax.dot_general(WT_ref[...], E, (((1,), (1,)), ((), ())),
                         precision=HI) + bcol_ref[...]
    lab = lab_ref[0]
    iota = lax.broadcasted_iota(jnp.int32, (C, B), 0)
    oh = (iota == lab).astype(jnp.float32)
    trT = transT_ref[...]

    @pl.when(t == 0)
    def _():
        alpha[...] = start_ref[...] + PT
        gold[...] = oh * (start_ref[...] + PT)
        ohprev[...] = oh
        loss_ref[...] = jnp.zeros((1, 1), jnp.float32)

    @pl.when(t > 0)
    def _():
        a = alpha[...]
        m = jnp.max(a, axis=0, keepdims=True)
        ea = jnp.exp(a - m)
        S = lax.dot_general(jnp.exp(trT), ea, (((1,), (0,)), ((), ())),
                            precision=HI)
        alpha[...] = m + jnp.log(S) + PT
        gtr = lax.dot_general(trT, ohprev[...], (((1,), (0,)), ((), ())),
                              precision=HI)
        gold[...] = gold[...] + oh * (PT + gtr)
        ohprev[...] = oh

    @pl.when(t == L - 1)
    def _():
        a2 = alpha[...] + end_ref[...]
        m2 = jnp.max(a2, axis=0, keepdims=True)
        logZ = m2 + jnp.log(jnp.sum(jnp.exp(a2 - m2), axis=0, keepdims=True))
        goldtot = gold[...] + oh * end_ref[...]
        grow = jnp.sum(goldtot, axis=0, keepdims=True)
        val = jnp.sum(grow - logZ, axis=1, keepdims=True)
        loss_ref[...] = -val


def _crf_call(embed_T, labels_T3, W, WT, transT, start2, end2, brow, bcol,
              interpret=False):
    L, B, D = embed_T.shape
    C = W.shape[1]
    body = functools.partial(_crf_body, L, B, C)
    logitsT, loss = pl.pallas_call(
        body,
        grid=(L,),
        in_specs=[
            pl.BlockSpec((1, B, D), lambda l: (l, 0, 0)),
            pl.BlockSpec((1, 1, B), lambda l: (l, 0, 0)),
            pl.BlockSpec((D, C), lambda l: (0, 0)),
            pl.BlockSpec((C, D), lambda l: (0, 0)),
            pl.BlockSpec((C, C), lambda l: (0, 0)),
            pl.BlockSpec((C, 1), lambda l: (0, 0)),
            pl.BlockSpec((C, 1), lambda l: (0, 0)),
            pl.BlockSpec((1, C), lambda l: (0, 0)),
            pl.BlockSpec((C, 1), lambda l: (0, 0)),
        ],
        out_specs=[
            pl.BlockSpec((1, B, C), lambda l: (l, 0, 0)),
            pl.BlockSpec((1, 1), lambda l: (0, 0)),
        ],
        out_shape=[
            jax.ShapeDtypeStruct((L, B, C), jnp.float32),
            jax.ShapeDtypeStruct((1, 1), jnp.float32),
        ],
        scratch_shapes=[
            pltpu.VMEM((C, B), jnp.float32),
            pltpu.VMEM((C, B), jnp.float32),
            pltpu.VMEM((C, B), jnp.float32),
        ],
        compiler_params=pltpu.CompilerParams(
            dimension_semantics=("arbitrary",)),
        interpret=interpret,
    )(embed_T, labels_T3, W, WT, transT, start2, end2, brow, bcol)
    return logitsT, loss


def kernel(x, labels, table, W_fc, b_fc, start_t, end_t, trans):
    B, L = x.shape
    V, D = table.shape
    C = W_fc.shape[1]
    xT = jnp.swapaxes(x, 0, 1).reshape(-1)
    embed_flat = _sc_gather(table, xT)
    embed_T = embed_flat.reshape(L, B, D)
    labels_T3 = jnp.swapaxes(labels, 0, 1).reshape(L, 1, B)
    logitsT, loss = _crf_call(
        embed_T, labels_T3,
        W_fc, jnp.swapaxes(W_fc, 0, 1), jnp.swapaxes(trans, 0, 1),
        start_t.reshape(C, 1), end_t.reshape(C, 1),
        b_fc.reshape(1, C), b_fc.reshape(C, 1),
    )
    logits = jnp.swapaxes(logitsT, 0, 1)
    return (logits, loss[0, 0])


# speedup vs baseline: 2.4390x; 2.4390x over previous
import functools

import jax
import jax.numpy as jnp
from jax import lax
from jax.experimental import pallas as pl
from jax.experimental.pallas import tpu as pltpu
from jax.experimental.pallas import tpu_sc as plsc


def _sc_gather(table, idx):
    V, D = table.shape
    N = idx.shape[0]
    info = plsc.get_sparse_core_info()
    NC, NS = info.num_cores, info.num_subcores
    NW = NC * NS
    assert N % NW == 0
    per_w = N // NW
    K = 128
    assert per_w % K == 0
    n_chunks = per_w // K
    NBUF = 5 if n_chunks % 5 == 0 else (4 if n_chunks % 4 == 0 else 2)
    assert n_chunks % NBUF == 0
    n_groups = n_chunks // NBUF

    mesh = plsc.VectorSubcoreMesh(core_axis_name="c", subcore_axis_name="s")

    @functools.partial(
        pl.kernel,
        mesh=mesh,
        compiler_params=pltpu.CompilerParams(use_tc_tiling_on_sc=False),
        out_type=jax.ShapeDtypeStruct((N, D), jnp.float32),
        scratch_types=(
            [pltpu.VMEM((per_w,), jnp.int32)]
            + [pltpu.VMEM((K, D), jnp.float32) for _ in range(NBUF)]
            + [pltpu.SemaphoreType.DMA for _ in range(2 * NBUF)]
        ),
    )
    def k(table_hbm, idx_hbm, out_hbm, *scr):
        idx_v = scr[0]
        bufs = scr[1:1 + NBUF]
        gsems = scr[1 + NBUF:1 + 2 * NBUF]
        wsems = scr[1 + 2 * NBUF:1 + 3 * NBUF]
        wid = lax.axis_index("s") * NC + lax.axis_index("c")
        base = wid * per_w
        pltpu.sync_copy(idx_hbm.at[pl.ds(base, per_w)], idx_v)

        def group(g, carry):
            off = g * (NBUF * K)
            gathers = []
            for b in range(NBUF):
                cp = pltpu.make_async_copy(
                    table_hbm.at[idx_v.at[pl.ds(off + b * K, K)]],
                    bufs[b], gsems[b])
                cp.start()
                gathers.append(cp)
            writes = []
            for b in range(NBUF):
                gathers[b].wait()
                wp = pltpu.make_async_copy(
                    bufs[b], out_hbm.at[pl.ds(base + off + b * K, K)],
                    wsems[b])
                wp.start()
                writes.append(wp)
            for b in range(NBUF):
                writes[b].wait()
            return carry

        lax.fori_loop(0, n_groups, group, 0, unroll=False)

    return k(table, idx)


def _crf_body(L, B, C,
              embed_ref, lab_ref, W_ref, WT_ref, transT_ref,
              start_ref, end_ref, brow_ref, bcol_ref,
              logits_ref, loss_ref,
              alpha, gold, ohprev):
    t = pl.program_id(0)
    E = embed_ref[0]
    HI = jax.lax.Precision.HIGHEST
    P = lax.dot_general(E, W_ref[...], (((1,), (0,)), ((), ())),
                        precision=HI) + brow_ref[...]
    logits_ref[0] = P
    PT = lax.dot_general(WT_ref[...], E, (((1,), (1,)), ((), ())),
                         precision=HI) + bcol_ref[...]
    lab = lab_ref[0]
    iota = lax.broadcasted_iota(jnp.int32, (C, B), 0)
    oh = (iota == lab).astype(jnp.float32)
    trT = transT_ref[...]

    @pl.when(t == 0)
    def _():
        alpha[...] = start_ref[...] + PT
        gold[...] = oh * (start_ref[...] + PT)
        ohprev[...] = oh
        loss_ref[...] = jnp.zeros((1, 1), jnp.float32)

    @pl.when(t > 0)
    def _():
        a = alpha[...]
        m = jnp.max(a, axis=0, keepdims=True)
        ea = jnp.exp(a - m)
        S = lax.dot_general(jnp.exp(trT), ea, (((1,), (0,)), ((), ())),
                            precision=HI)
        alpha[...] = m + jnp.log(S) + PT
        gtr = lax.dot_general(trT, ohprev[...], (((1,), (0,)), ((), ())),
                              precision=HI)
        gold[...] = gold[...] + oh * (PT + gtr)
        ohprev[...] = oh

    @pl.when(t == L - 1)
    def _():
        a2 = alpha[...] + end_ref[...]
        m2 = jnp.max(a2, axis=0, keepdims=True)
        logZ = m2 + jnp.log(jnp.sum(jnp.exp(a2 - m2), axis=0, keepdims=True))
        goldtot = gold[...] + oh * end_ref[...]
        grow = jnp.sum(goldtot, axis=0, keepdims=True)
        val = jnp.sum(grow - logZ, axis=1, keepdims=True)
        loss_ref[...] = -val


def _crf_call(embed_T, labels_T3, W, WT, transT, start2, end2, brow, bcol,
              interpret=False):
    L, B, D = embed_T.shape
    C = W.shape[1]
    body = functools.partial(_crf_body, L, B, C)
    logitsT, loss = pl.pallas_call(
        body,
        grid=(L,),
        in_specs=[
            pl.BlockSpec((1, B, D), lambda l: (l, 0, 0)),
            pl.BlockSpec((1, 1, B), lambda l: (l, 0, 0)),
            pl.BlockSpec((D, C), lambda l: (0, 0)),
            pl.BlockSpec((C, D), lambda l: (0, 0)),
            pl.BlockSpec((C, C), lambda l: (0, 0)),
            pl.BlockSpec((C, 1), lambda l: (0, 0)),
            pl.BlockSpec((C, 1), lambda l: (0, 0)),
            pl.BlockSpec((1, C), lambda l: (0, 0)),
            pl.BlockSpec((C, 1), lambda l: (0, 0)),
        ],
        out_specs=[
            pl.BlockSpec((1, B, C), lambda l: (l, 0, 0)),
            pl.BlockSpec((1, 1), lambda l: (0, 0)),
        ],
        out_shape=[
            jax.ShapeDtypeStruct((L, B, C), jnp.float32),
            jax.ShapeDtypeStruct((1, 1), jnp.float32),
        ],
        scratch_shapes=[
            pltpu.VMEM((C, B), jnp.float32),
            pltpu.VMEM((C, B), jnp.float32),
            pltpu.VMEM((C, B), jnp.float32),
        ],
        compiler_params=pltpu.CompilerParams(
            dimension_semantics=("arbitrary",)),
        interpret=interpret,
    )(embed_T, labels_T3, W, WT, transT, start2, end2, brow, bcol)
    return logitsT, loss


def kernel(x, labels, table, W_fc, b_fc, start_t, end_t, trans):
    B, L = x.shape
    V, D = table.shape
    C = W_fc.shape[1]
    xT = jnp.swapaxes(x, 0, 1).reshape(-1)
    embed_flat = _sc_gather(table, xT)
    embed_T = embed_flat.reshape(L, B, D)
    labels_T3 = jnp.swapaxes(labels, 0, 1).reshape(L, 1, B)
    logitsT, loss = _crf_call(
        embed_T, labels_T3,
        W_fc, jnp.swapaxes(W_fc, 0, 1), jnp.swapaxes(trans, 0, 1),
        start_t.reshape(C, 1), end_t.reshape(C, 1),
        b_fc.reshape(1, C), b_fc.reshape(C, 1),
    )
    logits = jnp.swapaxes(logitsT, 0, 1)
    return (logits, loss[0, 0])
